# Initial kernel scaffold; baseline (speedup 1.0000x reference)
import jax
import jax.numpy as jnp
from jax.experimental import pallas as pl


def kernel(points):
    raise NotImplementedError("write your pallas kernel here")


# speedup vs baseline: 4.3525x; 4.3525x over previous
import jax
import jax.numpy as jnp
from jax.experimental import pallas as pl

NEG_INF = float("-inf")


def _nms_body(x_ref, o_ref):
    x = x_ref[...]
    left = jnp.concatenate([jnp.full_like(x[:, :, :1], NEG_INF), x[:, :, :-1]], axis=2)
    right = jnp.concatenate([x[:, :, 1:], jnp.full_like(x[:, :, :1], NEG_INF)], axis=2)
    rowmax = jnp.maximum(jnp.maximum(left, x), right)
    up = jnp.concatenate([jnp.full_like(rowmax[:, :1, :], NEG_INF), rowmax[:, :-1, :]], axis=1)
    down = jnp.concatenate([rowmax[:, 1:, :], jnp.full_like(rowmax[:, :1, :], NEG_INF)], axis=1)
    hmax = jnp.maximum(jnp.maximum(up, rowmax), down)
    o_ref[...] = jnp.where(hmax == x, x, 0.0)


def kernel(points):
    n, c, h, w = points.shape
    x = points.reshape(n * c, h, w)
    blk = 8
    out = pl.pallas_call(
        _nms_body,
        grid=((n * c) // blk,),
        in_specs=[pl.BlockSpec((blk, h, w), lambda i: (i, 0, 0))],
        out_specs=pl.BlockSpec((blk, h, w), lambda i: (i, 0, 0)),
        out_shape=jax.ShapeDtypeStruct((n * c, h, w), points.dtype),
    )(x)
    return out.reshape(n, c, h, w)


# speedup vs baseline: 5.4008x; 1.2408x over previous
import jax
import jax.numpy as jnp
from jax.experimental import pallas as pl

NEG_INF = float("-inf")


def _nms_body(x_ref, o_ref):
    x = x_ref[...]
    left = jnp.concatenate([jnp.full_like(x[:, :, :1], NEG_INF), x[:, :, :-1]], axis=2)
    right = jnp.concatenate([x[:, :, 1:], jnp.full_like(x[:, :, :1], NEG_INF)], axis=2)
    rowmax = jnp.maximum(jnp.maximum(left, x), right)
    up = jnp.concatenate([jnp.full_like(rowmax[:, :1, :], NEG_INF), rowmax[:, :-1, :]], axis=1)
    down = jnp.concatenate([rowmax[:, 1:, :], jnp.full_like(rowmax[:, :1, :], NEG_INF)], axis=1)
    hmax = jnp.maximum(jnp.maximum(up, rowmax), down)
    o_ref[...] = jnp.where(hmax == x, x, 0.0)


def kernel(points):
    n, c, h, w = points.shape
    x = points.reshape(n * c, h, w)
    blk = 32
    out = pl.pallas_call(
        _nms_body,
        grid=((n * c) // blk,),
        in_specs=[pl.BlockSpec((blk, h, w), lambda i: (i, 0, 0))],
        out_specs=pl.BlockSpec((blk, h, w), lambda i: (i, 0, 0)),
        out_shape=jax.ShapeDtypeStruct((n * c, h, w), points.dtype),
    )(x)
    return out.reshape(n, c, h, w)
